# Initial kernel scaffold; baseline (speedup 1.0000x reference)
import jax
import jax.numpy as jnp
from jax.experimental import pallas as pl


def kernel(rpn_cls_prob, rpn_bbox_pred, im_info):
    raise NotImplementedError("write your pallas kernel here")


# speedup vs baseline: 33.1096x; 33.1096x over previous
import functools
import numpy as np
import jax
import jax.numpy as jnp
from jax.experimental import pallas as pl
from jax.experimental.pallas import tpu as pltpu

_N = 2
_A = 15
_H = 50
_W = 50
_PRE = 2000
_PREP = 2048
_POST = 1000
_POSTP = 1024
_THRESH = 0.7
_CLIP = float(np.log(1000.0 / 16.0))
_STRIDE = 16.0

_ANCHORS_NP = np.array([
    [-22.63, -11.31, 22.63, 11.31], [-16.0, -16.0, 16.0, 16.0], [-11.31, -22.63, 11.31, 22.63],
    [-45.25, -22.63, 45.25, 22.63], [-32.0, -32.0, 32.0, 32.0], [-22.63, -45.25, 22.63, 45.25],
    [-90.51, -45.25, 90.51, 45.25], [-64.0, -64.0, 64.0, 64.0], [-45.25, -90.51, 45.25, 90.51],
    [-181.02, -90.51, 181.02, 90.51], [-128.0, -128.0, 128.0, 128.0], [-90.51, -181.02, 90.51, 181.02],
    [-362.04, -181.02, 362.04, 181.02], [-256.0, -256.0, 256.0, 256.0], [-181.02, -362.04, 181.02, 362.04]],
    dtype=np.float32)


def _all_anchors_np():
    shift_x = np.arange(_W, dtype=np.float32) * _STRIDE
    shift_y = np.arange(_H, dtype=np.float32) * _STRIDE
    sx, sy = np.meshgrid(shift_x, shift_y)
    shifts = np.stack([sx.ravel(), sy.ravel(), sx.ravel(), sy.ravel()], axis=1)
    return (_ANCHORS_NP[None, :, :] + shifts[:, None, :]).reshape(-1, 4)


def _transform_clip_valid(anc, dl, vals, hh, ww, lane_idx):
    ax1, ay1, ax2, ay2 = anc
    dx, dy, dw, dh = dl
    widths = ax2 - ax1 + 1.0
    heights = ay2 - ay1 + 1.0
    ctr_x = ax1 + 0.5 * widths
    ctr_y = ay1 + 0.5 * heights
    dw = jnp.minimum(dw, _CLIP)
    dh = jnp.minimum(dh, _CLIP)
    pred_ctr_x = dx * widths + ctr_x
    pred_ctr_y = dy * heights + ctr_y
    pred_w = jnp.exp(dw) * widths
    pred_h = jnp.exp(dh) * heights
    x1 = pred_ctr_x - 0.5 * pred_w
    y1 = pred_ctr_y - 0.5 * pred_h
    x2 = pred_ctr_x + 0.5 * pred_w - 1.0
    y2 = pred_ctr_y + 0.5 * pred_h - 1.0
    x1 = jnp.clip(x1, 0.0, ww - 1.0)
    y1 = jnp.clip(y1, 0.0, hh - 1.0)
    x2 = jnp.clip(x2, 0.0, ww - 1.0)
    y2 = jnp.clip(y2, 0.0, hh - 1.0)
    ws = x2 - x1 + 1.0
    hs = y2 - y1 + 1.0
    x_ctr = x1 + ws / 2.0
    y_ctr = y1 + hs / 2.0
    valid = ((ws >= 0.0) & (hs >= 0.0) & (x_ctr < ww) & (y_ctr < hh)
             & (lane_idx < _PRE))
    area = ws * hs
    return x1, y1, x2, y2, area, valid.astype(jnp.float32)


def _nms_body(im_ref, ancr_ref, dlr_ref, valsr_ref, ancc_ref, dlc_ref,
              valsc_ref, out_ref, mask_ref, cols_ref, supp_ref):
    img = pl.program_id(0)
    hh = im_ref[0, 0, 0]
    ww = im_ref[0, 0, 1]

    lane = jax.lax.broadcasted_iota(jnp.int32, (1, _PREP), 1)
    anc_r = tuple(ancr_ref[0, k:k + 1, :] for k in range(4))
    dl_r = tuple(dlr_ref[0, k:k + 1, :] for k in range(4))
    x1r, y1r, x2r, y2r, area_r, valid_r = _transform_clip_valid(
        anc_r, dl_r, valsr_ref[0, 0:1, :], hh, ww, lane)

    sub = jax.lax.broadcasted_iota(jnp.int32, (_PREP, 1), 0)
    anc_c = tuple(ancc_ref[0, :, k:k + 1] for k in range(4))
    dl_c = tuple(dlc_ref[0, :, k:k + 1] for k in range(4))
    x1c, y1c, x2c, y2c, area_c, valid_c = _transform_clip_valid(
        anc_c, dl_c, valsc_ref[0, :, 0:1], hh, ww, sub)

    cols_ref[:, 0:1] = x1c
    cols_ref[:, 1:2] = y1c
    cols_ref[:, 2:3] = x2c
    cols_ref[:, 3:4] = y2c
    cols_ref[:, 4:5] = valsc_ref[0, :, 0:1]
    cols_ref[:, 5:6] = jnp.ones((_PREP, 1), jnp.float32)
    cols_ref[:, 6:8] = jnp.zeros((_PREP, 2), jnp.float32)

    for k in range(_PREP // 128):
        rs = slice(k * 128, (k + 1) * 128)
        bx1, by1 = x1c[rs, :], y1c[rs, :]
        bx2, by2 = x2c[rs, :], y2c[rs, :]
        barea = area_c[rs, :]
        xx1 = jnp.maximum(bx1, x1r)
        yy1 = jnp.maximum(by1, y1r)
        xx2 = jnp.minimum(bx2, x2r)
        yy2 = jnp.minimum(by2, y2r)
        w = jnp.maximum(0.0, xx2 - xx1 + 1.0)
        h = jnp.maximum(0.0, yy2 - yy1 + 1.0)
        inter = w * h
        ovr = inter / (barea + area_r - inter)
        ridx = jax.lax.broadcasted_iota(jnp.int32, (128, _PREP), 0) + k * 128
        m = (ovr > _THRESH) & (ridx != lane)
        mask_ref[rs, :] = m.astype(jnp.float32)

    supp_ref[0:1, :] = 1.0 - valid_r

    def body(i, _):
        s = jnp.sum(jnp.where(lane == i, supp_ref[0:1, :], 0.0))

        @pl.when(s == 0.0)
        def _():
            supp_ref[0:1, :] = jnp.maximum(supp_ref[0:1, :],
                                           mask_ref[pl.ds(i, 1), :])
        return 0

    jax.lax.fori_loop(0, _PRE, body, 0, unroll=False)

    keep_r = jnp.where(supp_ref[0:1, :] == 0.0, 1.0, 0.0)

    tri = (jax.lax.broadcasted_iota(jnp.int32, (_PREP, _PREP), 0)
           < jax.lax.broadcasted_iota(jnp.int32, (_PREP, _PREP), 1))
    pos_r = jax.lax.dot_general(
        keep_r, tri.astype(jnp.float32), (((1,), (0,)), ((), ())),
        preferred_element_type=jnp.float32)

    prow = jax.lax.broadcasted_iota(jnp.int32, (_POSTP, 1), 0).astype(jnp.float32)
    P = jnp.where((pos_r == prow) & (keep_r > 0.0), 1.0, 0.0)
    packed = jax.lax.dot_general(
        P, cols_ref[:, :], (((1,), (0,)), ((), ())),
        preferred_element_type=jnp.float32)

    out_ref[0, :, 0:1] = img.astype(jnp.float32) * packed[:, 5:6]
    out_ref[0, :, 1:5] = packed[:, 0:4]
    out_ref[0, :, 5:6] = packed[:, 4:5]
    out_ref[0, :, 6:8] = jnp.zeros((_POSTP, 2), jnp.float32)


@jax.jit
def kernel(rpn_cls_prob, rpn_bbox_pred, im_info):
    scores = jnp.transpose(rpn_cls_prob, (0, 2, 3, 1)).reshape(_N, -1)
    deltas = jnp.transpose(
        rpn_bbox_pred.reshape(_N, _A, 4, _H, _W), (0, 3, 4, 1, 2)
    ).reshape(_N, _H * _W * _A, 4)
    all_anchors = jnp.asarray(_all_anchors_np())

    vals, order = jax.lax.top_k(scores, _PRE)
    anc = all_anchors[order]
    dl = jnp.take_along_axis(deltas, order[:, :, None], axis=1)

    pad = ((0, 0), (0, _PREP - _PRE), (0, 0))
    anc_p = jnp.pad(anc, pad)
    dl_p = jnp.pad(dl, pad)
    vals_p = jnp.pad(vals, ((0, 0), (0, _PREP - _PRE)))

    anc_rt = jnp.transpose(anc_p, (0, 2, 1))
    dl_rt = jnp.transpose(dl_p, (0, 2, 1))
    vals_r = vals_p[:, None, :]
    vals_c = vals_p[:, :, None]

    out = pl.pallas_call(
        _nms_body,
        grid=(_N,),
        in_specs=[
            pl.BlockSpec((1, 1, 3), lambda i: (i, 0, 0),
                         memory_space=pltpu.SMEM),
            pl.BlockSpec((1, 4, _PREP), lambda i: (i, 0, 0)),
            pl.BlockSpec((1, 4, _PREP), lambda i: (i, 0, 0)),
            pl.BlockSpec((1, 1, _PREP), lambda i: (i, 0, 0)),
            pl.BlockSpec((1, _PREP, 4), lambda i: (i, 0, 0)),
            pl.BlockSpec((1, _PREP, 4), lambda i: (i, 0, 0)),
            pl.BlockSpec((1, _PREP, 1), lambda i: (i, 0, 0)),
        ],
        out_specs=pl.BlockSpec((1, _POSTP, 8), lambda i: (i, 0, 0)),
        out_shape=jax.ShapeDtypeStruct((_N, _POSTP, 8), jnp.float32),
        scratch_shapes=[
            pltpu.VMEM((_PREP, _PREP), jnp.float32),
            pltpu.VMEM((_PREP, 8), jnp.float32),
            pltpu.VMEM((1, _PREP), jnp.float32),
        ],
        compiler_params=pltpu.CompilerParams(
            dimension_semantics=("arbitrary",)),
    )(im_info[:, None, :], anc_rt, dl_rt, vals_r, anc_p, dl_p, vals_c)

    rois = out[:, :_POST, 0:5].reshape(_N * _POST, 5)
    probs = out[:, :_POST, 5:6].reshape(_N * _POST, 1)
    return rois, probs


# speedup vs baseline: 41.8624x; 1.2644x over previous
import functools
import numpy as np
import jax
import jax.numpy as jnp
from jax.experimental import pallas as pl
from jax.experimental.pallas import tpu as pltpu

_N = 2
_A = 15
_H = 50
_W = 50
_PRE = 2000
_PREP = 2048
_POST = 1000
_POSTP = 1024
_THRESH = 0.7
_CLIP = float(np.log(1000.0 / 16.0))
_STRIDE = 16.0

_ANCHORS_NP = np.array([
    [-22.63, -11.31, 22.63, 11.31], [-16.0, -16.0, 16.0, 16.0], [-11.31, -22.63, 11.31, 22.63],
    [-45.25, -22.63, 45.25, 22.63], [-32.0, -32.0, 32.0, 32.0], [-22.63, -45.25, 22.63, 45.25],
    [-90.51, -45.25, 90.51, 45.25], [-64.0, -64.0, 64.0, 64.0], [-45.25, -90.51, 45.25, 90.51],
    [-181.02, -90.51, 181.02, 90.51], [-128.0, -128.0, 128.0, 128.0], [-90.51, -181.02, 90.51, 181.02],
    [-362.04, -181.02, 362.04, 181.02], [-256.0, -256.0, 256.0, 256.0], [-181.02, -362.04, 181.02, 362.04]],
    dtype=np.float32)


def _all_anchors_np():
    shift_x = np.arange(_W, dtype=np.float32) * _STRIDE
    shift_y = np.arange(_H, dtype=np.float32) * _STRIDE
    sx, sy = np.meshgrid(shift_x, shift_y)
    shifts = np.stack([sx.ravel(), sy.ravel(), sx.ravel(), sy.ravel()], axis=1)
    return (_ANCHORS_NP[None, :, :] + shifts[:, None, :]).reshape(-1, 4)


def _transform_clip_valid(anc, dl, vals, hh, ww, lane_idx):
    ax1, ay1, ax2, ay2 = anc
    dx, dy, dw, dh = dl
    widths = ax2 - ax1 + 1.0
    heights = ay2 - ay1 + 1.0
    ctr_x = ax1 + 0.5 * widths
    ctr_y = ay1 + 0.5 * heights
    dw = jnp.minimum(dw, _CLIP)
    dh = jnp.minimum(dh, _CLIP)
    pred_ctr_x = dx * widths + ctr_x
    pred_ctr_y = dy * heights + ctr_y
    pred_w = jnp.exp(dw) * widths
    pred_h = jnp.exp(dh) * heights
    x1 = pred_ctr_x - 0.5 * pred_w
    y1 = pred_ctr_y - 0.5 * pred_h
    x2 = pred_ctr_x + 0.5 * pred_w - 1.0
    y2 = pred_ctr_y + 0.5 * pred_h - 1.0
    x1 = jnp.clip(x1, 0.0, ww - 1.0)
    y1 = jnp.clip(y1, 0.0, hh - 1.0)
    x2 = jnp.clip(x2, 0.0, ww - 1.0)
    y2 = jnp.clip(y2, 0.0, hh - 1.0)
    ws = x2 - x1 + 1.0
    hs = y2 - y1 + 1.0
    x_ctr = x1 + ws / 2.0
    y_ctr = y1 + hs / 2.0
    valid = ((ws >= 0.0) & (hs >= 0.0) & (x_ctr < ww) & (y_ctr < hh)
             & (lane_idx < _PRE))
    area = ws * hs
    return x1, y1, x2, y2, area, valid.astype(jnp.float32)


def _nms_body(im_ref, ancr_ref, dlr_ref, valsr_ref, ancc_ref, dlc_ref,
              valsc_ref, out_ref, mask_ref, cols_ref, supp_ref):
    img = pl.program_id(0)
    hh = im_ref[0, 0, 0]
    ww = im_ref[0, 0, 1]

    lane = jax.lax.broadcasted_iota(jnp.int32, (1, _PREP), 1)
    anc_r = tuple(ancr_ref[0, k:k + 1, :] for k in range(4))
    dl_r = tuple(dlr_ref[0, k:k + 1, :] for k in range(4))
    x1r, y1r, x2r, y2r, area_r, valid_r = _transform_clip_valid(
        anc_r, dl_r, valsr_ref[0, 0:1, :], hh, ww, lane)

    sub = jax.lax.broadcasted_iota(jnp.int32, (_PREP, 1), 0)
    anc_c = tuple(ancc_ref[0, :, k:k + 1] for k in range(4))
    dl_c = tuple(dlc_ref[0, :, k:k + 1] for k in range(4))
    x1c, y1c, x2c, y2c, area_c, valid_c = _transform_clip_valid(
        anc_c, dl_c, valsc_ref[0, :, 0:1], hh, ww, sub)

    cols_ref[:, 0:1] = x1c
    cols_ref[:, 1:2] = y1c
    cols_ref[:, 2:3] = x2c
    cols_ref[:, 3:4] = y2c
    cols_ref[:, 4:5] = valsc_ref[0, :, 0:1]
    cols_ref[:, 5:6] = jnp.ones((_PREP, 1), jnp.float32)
    cols_ref[:, 6:8] = jnp.zeros((_PREP, 2), jnp.float32)

    for k in range(_PREP // 128):
        rs = slice(k * 128, (k + 1) * 128)
        bx1, by1 = x1c[rs, :], y1c[rs, :]
        bx2, by2 = x2c[rs, :], y2c[rs, :]
        barea = area_c[rs, :]
        xx1 = jnp.maximum(bx1, x1r)
        yy1 = jnp.maximum(by1, y1r)
        xx2 = jnp.minimum(bx2, x2r)
        yy2 = jnp.minimum(by2, y2r)
        w = jnp.maximum(0.0, xx2 - xx1 + 1.0)
        h = jnp.maximum(0.0, yy2 - yy1 + 1.0)
        inter = w * h
        ovr = inter / (barea + area_r - inter)
        ridx = jax.lax.broadcasted_iota(jnp.int32, (128, _PREP), 0) + k * 128
        m = (ovr > _THRESH) & (ridx != lane)
        mask_ref[rs, :] = m.astype(jnp.float32)

    supp_ref[0:1, :] = 1.0 - valid_r

    def cond(carry):
        i, cnt = carry
        return (i < _PRE) & (cnt < _POST)

    def body(carry):
        i, cnt = carry
        s = jnp.sum(jnp.where(lane == i, supp_ref[0:1, :], 0.0))

        @pl.when(s == 0.0)
        def _():
            supp_ref[0:1, :] = jnp.maximum(supp_ref[0:1, :],
                                           mask_ref[pl.ds(i, 1), :])
        return i + 1, cnt + jnp.where(s == 0.0, 1, 0)

    jax.lax.while_loop(cond, body, (0, 0))

    keep_r = jnp.where(supp_ref[0:1, :] == 0.0, 1.0, 0.0)

    tri = (jax.lax.broadcasted_iota(jnp.int32, (_PREP, _PREP), 0)
           < jax.lax.broadcasted_iota(jnp.int32, (_PREP, _PREP), 1))
    pos_r = jax.lax.dot_general(
        keep_r, tri.astype(jnp.float32), (((1,), (0,)), ((), ())),
        preferred_element_type=jnp.float32)

    prow = jax.lax.broadcasted_iota(jnp.int32, (_POSTP, 1), 0).astype(jnp.float32)
    P = jnp.where((pos_r == prow) & (keep_r > 0.0), 1.0, 0.0)
    packed = jax.lax.dot_general(
        P, cols_ref[:, :], (((1,), (0,)), ((), ())),
        preferred_element_type=jnp.float32)

    out_ref[0, :, 0:1] = img.astype(jnp.float32) * packed[:, 5:6]
    out_ref[0, :, 1:5] = packed[:, 0:4]
    out_ref[0, :, 5:6] = packed[:, 4:5]
    out_ref[0, :, 6:8] = jnp.zeros((_POSTP, 2), jnp.float32)


@jax.jit
def kernel(rpn_cls_prob, rpn_bbox_pred, im_info):
    scores = jnp.transpose(rpn_cls_prob, (0, 2, 3, 1)).reshape(_N, -1)
    deltas = jnp.transpose(
        rpn_bbox_pred.reshape(_N, _A, 4, _H, _W), (0, 3, 4, 1, 2)
    ).reshape(_N, _H * _W * _A, 4)
    all_anchors = jnp.asarray(_all_anchors_np())

    vals, order = jax.lax.top_k(scores, _PRE)
    anc = all_anchors[order]
    dl = jnp.take_along_axis(deltas, order[:, :, None], axis=1)

    pad = ((0, 0), (0, _PREP - _PRE), (0, 0))
    anc_p = jnp.pad(anc, pad)
    dl_p = jnp.pad(dl, pad)
    vals_p = jnp.pad(vals, ((0, 0), (0, _PREP - _PRE)))

    anc_rt = jnp.transpose(anc_p, (0, 2, 1))
    dl_rt = jnp.transpose(dl_p, (0, 2, 1))
    vals_r = vals_p[:, None, :]
    vals_c = vals_p[:, :, None]

    out = pl.pallas_call(
        _nms_body,
        grid=(_N,),
        in_specs=[
            pl.BlockSpec((1, 1, 3), lambda i: (i, 0, 0),
                         memory_space=pltpu.SMEM),
            pl.BlockSpec((1, 4, _PREP), lambda i: (i, 0, 0)),
            pl.BlockSpec((1, 4, _PREP), lambda i: (i, 0, 0)),
            pl.BlockSpec((1, 1, _PREP), lambda i: (i, 0, 0)),
            pl.BlockSpec((1, _PREP, 4), lambda i: (i, 0, 0)),
            pl.BlockSpec((1, _PREP, 4), lambda i: (i, 0, 0)),
            pl.BlockSpec((1, _PREP, 1), lambda i: (i, 0, 0)),
        ],
        out_specs=pl.BlockSpec((1, _POSTP, 8), lambda i: (i, 0, 0)),
        out_shape=jax.ShapeDtypeStruct((_N, _POSTP, 8), jnp.float32),
        scratch_shapes=[
            pltpu.VMEM((_PREP, _PREP), jnp.float32),
            pltpu.VMEM((_PREP, 8), jnp.float32),
            pltpu.VMEM((1, _PREP), jnp.float32),
        ],
        compiler_params=pltpu.CompilerParams(
            dimension_semantics=("parallel",)),
    )(im_info[:, None, :], anc_rt, dl_rt, vals_r, anc_p, dl_p, vals_c)

    rois = out[:, :_POST, 0:5].reshape(_N * _POST, 5)
    probs = out[:, :_POST, 5:6].reshape(_N * _POST, 1)
    return rois, probs
